# baseline (speedup 1.0000x reference)
import jax
import jax.numpy as jnp
from jax import lax
from jax.experimental import pallas as pl
from jax.experimental.pallas import tpu as pltpu
import jax.experimental.pallas.tpu_sc as plsc

BATCH = 16384
D = 16
NC = 2
NS = 16
L = 16
NW = NC * NS
BPW = BATCH // NW
CHUNK = 128
NCHUNK = BPW // CHUNK
ROWS_PER_BIG = 128 // D


def _body(uid_hbm, iid_hbm, ut_hbm, it_hbm, w_hbm, b_hbm, out_hbm,
          idx_u, idx_i, bidx_u, bidx_i,
          u_big0, u_big1, i_big0, i_big1, out_v, wv, bv,
          sem0, sem1):
  c = lax.axis_index("c")
  s = lax.axis_index("s")
  wid = s * NC + c
  base = wid * BPW

  pltpu.sync_copy(uid_hbm.at[pl.ds(base, BPW)], idx_u)
  pltpu.sync_copy(iid_hbm.at[pl.ds(base, BPW)], idx_i)
  pltpu.sync_copy(w_hbm, wv)
  pltpu.sync_copy(b_hbm, bv)

  def bidx_block(t, _):
    sl = pl.ds(t * L, L)
    bidx_u[sl] = lax.shift_right_logical(idx_u[sl], 3)
    bidx_i[sl] = lax.shift_right_logical(idx_i[sl], 3)
    return 0
  lax.fori_loop(0, BPW // L, bidx_block, 0)

  u_bigs = (u_big0, u_big1)
  i_bigs = (i_big0, i_big1)
  sems = (sem0, sem1)

  def fire(j):
    sl = pl.ds(j * CHUNK, CHUNK)
    return (
        pltpu.async_copy(ut_hbm.at[bidx_u.at[sl]], u_bigs[j % 2], sems[j % 2]),
        pltpu.async_copy(it_hbm.at[bidx_i.at[sl]], i_bigs[j % 2], sems[j % 2]),
    )

  w_s = wv[...]
  b_s = bv[...]
  iota = lax.iota(jnp.int32, L)

  cps = fire(0)
  for j in range(NCHUNK):
    nxt = fire(j + 1) if j + 1 < NCHUNK else None
    for cp in cps:
      cp.wait()
    u_big = u_bigs[j % 2]
    i_big = i_bigs[j % 2]

    def group(g, _):
      off = j * CHUNK + g * L
      sl = pl.ds(off, L)
      sub_u = lax.shift_left(jnp.bitwise_and(idx_u[sl], ROWS_PER_BIG - 1), 4)
      sub_i = lax.shift_left(jnp.bitwise_and(idx_i[sl], ROWS_PER_BIG - 1), 4)
      rows = g * L + iota
      acc = None
      for d in range(D):
        cu = plsc.load_gather(u_big, [rows, sub_u + d])
        ci = plsc.load_gather(i_big, [rows, sub_i + d])
        prod = cu * ci
        acc = prod if acc is None else acc + prod
      out_v[sl] = acc * w_s + b_s
      return 0

    lax.fori_loop(0, CHUNK // L, group, 0)
    cps = nxt

  pltpu.sync_copy(out_v, out_hbm.at[pl.ds(base, BPW)])


@jax.jit
def _run(userID, ItemID, user_table, item_table, w, b):
  mesh = plsc.VectorSubcoreMesh(core_axis_name="c", subcore_axis_name="s")
  f = pl.kernel(
      _body,
      out_type=jax.ShapeDtypeStruct((BATCH,), jnp.float32),
      mesh=mesh,
      scratch_types=[
          pltpu.VMEM((BPW,), jnp.int32),
          pltpu.VMEM((BPW,), jnp.int32),
          pltpu.VMEM((BPW,), jnp.int32),
          pltpu.VMEM((BPW,), jnp.int32),
          pltpu.VMEM((CHUNK, 128), jnp.float32),
          pltpu.VMEM((CHUNK, 128), jnp.float32),
          pltpu.VMEM((CHUNK, 128), jnp.float32),
          pltpu.VMEM((CHUNK, 128), jnp.float32),
          pltpu.VMEM((BPW,), jnp.float32),
          pltpu.VMEM((L,), jnp.float32),
          pltpu.VMEM((L,), jnp.float32),
          pltpu.SemaphoreType.DMA,
          pltpu.SemaphoreType.DMA,
      ],
      compiler_params=pltpu.CompilerParams(needs_layout_passes=False),
  )
  return f(userID, ItemID, user_table, item_table, w, b)


def kernel(userID, ItemID, user_table, item_table, w, b):
  w16 = jnp.broadcast_to(jnp.reshape(w, (1,)), (L,))
  b16 = jnp.broadcast_to(jnp.reshape(b, (1,)), (L,))
  ut2 = jnp.reshape(user_table, (-1, 128))
  it2 = jnp.reshape(item_table, (-1, 128))
  return _run(userID.astype(jnp.int32), ItemID.astype(jnp.int32),
              ut2, it2, w16, b16)
